# baseline (speedup 1.0000x reference)
import functools

import jax
import jax.numpy as jnp
from jax import lax
from jax.experimental import pallas as pl
from jax.experimental.pallas import tpu as pltpu
from jax.experimental.pallas import tpu_sc as plsc

MIN_POS = 0.0
MAX_POS = 1.0
N_POS = 100000
N_HEADS = 16

NC = 2
NS = 16
NW = NC * NS
L = 16

JB = 8
IB = 128
TH = N_HEADS // JB


def _make_sc_kernel(n_i, n_j):
    ti_count = n_i // IB
    units = (n_j // JB) * ti_count
    per_w = units // NW
    mesh = plsc.VectorSubcoreMesh(
        core_axis_name="c", subcore_axis_name="s", num_cores=NC, num_subcores=NS
    )

    @functools.partial(
        pl.kernel,
        out_type=(
            jax.ShapeDtypeStruct((n_j, TH, ti_count, JB * IB), jnp.float32),
            jax.ShapeDtypeStruct((n_j // JB, ti_count, JB, IB), jnp.int32),
        ),
        mesh=mesh,
        scratch_types=[
            pltpu.VMEM((2, JB, IB), jnp.float32),
            pltpu.VMEM((2, JB, IB), jnp.int32),
            pltpu.VMEM((2, JB * IB, N_HEADS), jnp.float32),
            pltpu.VMEM((2, JB, TH, JB * IB), jnp.float32),
            pltpu.SemaphoreType.DMA,
            pltpu.SemaphoreType.DMA,
            pltpu.SemaphoreType.DMA,
            pltpu.SemaphoreType.DMA,
            pltpu.SemaphoreType.DMA,
            pltpu.SemaphoreType.DMA,
            pltpu.SemaphoreType.DMA,
        ],
        compiler_params=pltpu.CompilerParams(
            use_tc_tiling_on_sc=False, needs_layout_passes=False
        ),
    )
    def body(
        coord_hbm, table_hbm, emb_hbm, idx_hbm,
        coord_v, idx_v, rows_v, out_v,
        sem_c, sem_g0, sem_g1, sem_i0, sem_i1, sem_o0, sem_o1,
    ):
        wid = lax.axis_index("s") * NC + lax.axis_index("c")
        wstart = wid * per_w
        scale = jnp.float32(N_POS / (MAX_POS - MIN_POS))
        lane = lax.iota(jnp.int32, L)
        hm_off = (lane & 7) * IB
        thv = lane >> 3
        jmv = [jnp.full((L,), jm, jnp.int32) for jm in range(JB)]
        sem_g = (sem_g0, sem_g1)
        sem_i = (sem_i0, sem_i1)
        sem_o = (sem_o0, sem_o1)

        def unit_pos(urel):
            g = wstart + urel
            j8 = g // ti_count
            ti = g % ti_count
            return pl.multiple_of(j8 * JB, JB), j8, ti

        def compute_idx_and_fire(urel, b):
            _, j8, ti = unit_pos(urel)
            for jm in range(JB):
                for c in range(IB // L):
                    v = coord_v[b, jm, pl.ds(c * L, L)]
                    pos = jnp.clip((v - MIN_POS) * scale, 0.0, float(N_POS))
                    idx_v[b, jm, pl.ds(c * L, L)] = pos.astype(jnp.int32)
            pltpu.async_copy(idx_v.at[b], idx_hbm.at[j8, ti], sem_i[b])
            for jm in range(JB):
                pltpu.async_copy(
                    table_hbm.at[idx_v.at[b, jm]],
                    rows_v.at[b, pl.ds(jm * IB, IB)],
                    sem_g[b],
                )

        def fire_coord(urel, b):
            j0, _, ti = unit_pos(urel)
            i0 = pl.multiple_of(ti * IB, IB)
            pltpu.async_copy(
                coord_hbm.at[pl.ds(j0, JB), pl.ds(i0, IB)],
                coord_v.at[b],
                sem_c,
            )

        def drain(src, dst, sem):
            pltpu.make_async_copy(src, dst, sem).wait()

        def drain_coord(b):
            drain(coord_hbm.at[pl.ds(0, JB), pl.ds(0, IB)], coord_v.at[b], sem_c)

        def drain_idx(b):
            drain(idx_v.at[b], idx_hbm.at[0, 0], sem_i[b])

        def drain_gathers(b):
            drain(table_hbm.at[pl.ds(0, JB * IB)], rows_v.at[b], sem_g[b])

        def drain_out(b):
            drain(out_v.at[b], emb_hbm.at[pl.ds(0, JB), :, 0], sem_o[b])

        def transpose_and_fire_out(urel, b):
            rows_b = rows_v.at[b]
            out_b = out_v.at[b]

            def tbody(ii, tc):
                off = hm_off + ii
                for jm in range(JB):
                    vec = rows_b[jm * IB + ii]
                    plsc.store_scatter(out_b, [jmv[jm], thv, off], vec)
                return tc

            lax.fori_loop(0, IB, tbody, 0, unroll=False)

            j0, _, ti = unit_pos(urel)
            pltpu.async_copy(out_b, emb_hbm.at[pl.ds(j0, JB), :, ti], sem_o[b])

        def unit_serial(urel, carry):
            j0s, _, tis = unit_pos(urel)
            pltpu.sync_copy(
                coord_hbm.at[pl.ds(j0s, JB), pl.ds(pl.multiple_of(tis * IB, IB), IB)],
                coord_v.at[0],
            )
            compute_idx_and_fire(urel, 0)
            drain_idx(0)
            drain_gathers(0)
            transpose_and_fire_out(urel, 0)
            drain_out(0)
            return carry

        lax.fori_loop(0, per_w, unit_serial, 0, unroll=False)
        return

        j0p, _, tip = unit_pos(0)
        pltpu.sync_copy(
            coord_hbm.at[pl.ds(j0p, JB), pl.ds(pl.multiple_of(tip * IB, IB), IB)],
            coord_v.at[0],
        )
        compute_idx_and_fire(0, 0)
        fire_coord(1, 1)

        def stage(t, u, cur, is_a):
            nxt = 1 - cur
            u1 = u + 1

            def front_end():
                drain_coord(nxt)
                if is_a:
                    @pl.when(t >= 1)
                    def _():
                        drain_idx(nxt)
                else:
                    drain_idx(nxt)
                compute_idx_and_fire(u1, nxt)

                @pl.when(u + 2 < per_w)
                def _():
                    fire_coord(u + 2, cur)

            if is_a:
                front_end()
            else:
                @pl.when(u1 < per_w)
                def _():
                    front_end()

            @pl.when(t >= 1)
            def _():
                drain_out(cur)
            drain_gathers(cur)
            transpose_and_fire_out(u, cur)

        def pair_body(t, carry):
            stage(t, 2 * t, 0, True)
            stage(t, 2 * t + 1, 1, False)
            return carry

        lax.fori_loop(0, per_w // 2, pair_body, 0, unroll=False)

        drain_out(0)
        drain_out(1)
        drain_idx(0)
        drain_idx(1)

    return body


def kernel(coord, embeddings_table):
    n_i, n_j = coord.shape
    coord_t = coord.T
    emb4, idx4 = _make_sc_kernel(n_i, n_j)(coord_t, embeddings_table)
    emb5 = emb4.reshape(n_j, TH, n_i // IB, JB, IB)
    emb = jnp.transpose(emb5, (2, 4, 0, 1, 3)).reshape(n_i, n_j, N_HEADS)
    idx = jnp.transpose(idx4, (1, 3, 0, 2)).reshape(n_i, n_j)
    return emb, idx


# speedup vs baseline: 1.3062x; 1.3062x over previous
import functools

import jax
import jax.numpy as jnp
from jax import lax
from jax.experimental import pallas as pl
from jax.experimental.pallas import tpu as pltpu
from jax.experimental.pallas import tpu_sc as plsc

MIN_POS = 0.0
MAX_POS = 1.0
N_POS = 100000
N_HEADS = 16

NC = 2
NS = 16
NW = NC * NS
L = 16

JB = 8
IB = 128
TH = N_HEADS // JB


def _make_sc_kernel(n_i, n_j):
    ti_count = n_i // IB
    units = (n_j // JB) * ti_count
    per_w = units // NW
    mesh = plsc.VectorSubcoreMesh(
        core_axis_name="c", subcore_axis_name="s", num_cores=NC, num_subcores=NS
    )

    @functools.partial(
        pl.kernel,
        out_type=(
            jax.ShapeDtypeStruct((n_j, TH, ti_count, JB * IB), jnp.float32),
            jax.ShapeDtypeStruct((n_j // JB, ti_count, JB, IB), jnp.int32),
        ),
        mesh=mesh,
        scratch_types=[
            pltpu.VMEM((2, JB, IB), jnp.float32),
            pltpu.VMEM((2, JB, IB), jnp.int32),
            pltpu.VMEM((2, JB * IB, N_HEADS), jnp.float32),
            pltpu.VMEM((2, JB, TH, JB * IB), jnp.float32),
            pltpu.SemaphoreType.DMA,
            pltpu.SemaphoreType.DMA,
            pltpu.SemaphoreType.DMA,
            pltpu.SemaphoreType.DMA,
            pltpu.SemaphoreType.DMA,
            pltpu.SemaphoreType.DMA,
            pltpu.SemaphoreType.DMA,
        ],
        compiler_params=pltpu.CompilerParams(
            use_tc_tiling_on_sc=False, needs_layout_passes=False
        ),
    )
    def body(
        coord_hbm, table_hbm, emb_hbm, idx_hbm,
        coord_v, idx_v, rows_v, out_v,
        sem_c, sem_g0, sem_g1, sem_i0, sem_i1, sem_o0, sem_o1,
    ):
        wid = lax.axis_index("s") * NC + lax.axis_index("c")
        wstart = wid * per_w
        scale = jnp.float32(N_POS / (MAX_POS - MIN_POS))
        lane = lax.iota(jnp.int32, L)
        hm_off = (lane & 7) * IB
        thv = lane >> 3
        jmv = [jnp.full((L,), jm, jnp.int32) for jm in range(JB)]
        sem_g = (sem_g0, sem_g1)
        sem_i = (sem_i0, sem_i1)
        sem_o = (sem_o0, sem_o1)

        def unit_pos(urel):
            g = wstart + urel
            j8 = g // ti_count
            ti = g % ti_count
            return pl.multiple_of(j8 * JB, JB), j8, ti

        def compute_idx_and_fire(urel, b):
            _, j8, ti = unit_pos(urel)
            for jm in range(JB):
                for c in range(IB // L):
                    v = coord_v[b, jm, pl.ds(c * L, L)]
                    pos = jnp.clip((v - MIN_POS) * scale, 0.0, float(N_POS))
                    idx_v[b, jm, pl.ds(c * L, L)] = pos.astype(jnp.int32)
            pltpu.async_copy(idx_v.at[b], idx_hbm.at[j8, ti], sem_i[b])
            for jm in range(JB):
                pltpu.async_copy(
                    table_hbm.at[idx_v.at[b, jm]],
                    rows_v.at[b, pl.ds(jm * IB, IB)],
                    sem_g[b],
                )

        def fire_coord(urel, b):
            j0, _, ti = unit_pos(urel)
            i0 = pl.multiple_of(ti * IB, IB)
            pltpu.async_copy(
                coord_hbm.at[pl.ds(j0, JB), pl.ds(i0, IB)],
                coord_v.at[b],
                sem_c,
            )

        def drain(src, dst, sem):
            pltpu.make_async_copy(src, dst, sem).wait()

        def drain_coord(b):
            drain(coord_hbm.at[pl.ds(0, JB), pl.ds(0, IB)], coord_v.at[b], sem_c)

        def drain_idx(b):
            drain(idx_v.at[b], idx_hbm.at[0, 0], sem_i[b])

        def drain_gathers(b):
            drain(table_hbm.at[pl.ds(0, JB * IB)], rows_v.at[b], sem_g[b])

        def drain_out(b):
            drain(out_v.at[b], emb_hbm.at[pl.ds(0, JB), :, 0], sem_o[b])

        def transpose_and_fire_out(urel, b):
            rows_b = rows_v.at[b]
            out_b = out_v.at[b]

            def tbody(ii, tc):
                off = hm_off + ii
                for jm in range(JB):
                    vec = rows_b[jm * IB + ii]
                    plsc.store_scatter(out_b, [jmv[jm], thv, off], vec)
                return tc

            lax.fori_loop(0, IB, tbody, 0, unroll=False)

            j0, _, ti = unit_pos(urel)
            pltpu.async_copy(out_b, emb_hbm.at[pl.ds(j0, JB), :, ti], sem_o[b])

        j0p, _, tip = unit_pos(0)
        pltpu.sync_copy(
            coord_hbm.at[pl.ds(j0p, JB), pl.ds(pl.multiple_of(tip * IB, IB), IB)],
            coord_v.at[0],
        )
        compute_idx_and_fire(0, 0)
        fire_coord(1, 1)

        def stage(t, u, cur, is_a):
            nxt = 1 - cur
            u1 = u + 1

            def front_end():
                drain_coord(nxt)
                if is_a:
                    @pl.when(t >= 1)
                    def _():
                        drain_idx(nxt)
                else:
                    drain_idx(nxt)
                compute_idx_and_fire(u1, nxt)

                @pl.when(u + 2 < per_w)
                def _():
                    fire_coord(u + 2, cur)

            if is_a:
                front_end()
            else:
                @pl.when(u1 < per_w)
                def _():
                    front_end()

            @pl.when(t >= 1)
            def _():
                drain_out(cur)
            drain_gathers(cur)
            transpose_and_fire_out(u, cur)

        def pair_body(t, carry):
            stage(t, 2 * t, 0, True)
            stage(t, 2 * t + 1, 1, False)
            return carry

        lax.fori_loop(0, per_w // 2, pair_body, 0, unroll=False)

        drain_out(0)
        drain_out(1)
        drain_idx(0)
        drain_idx(1)

    return body


def kernel(coord, embeddings_table):
    n_i, n_j = coord.shape
    coord_t = coord.T
    emb4, idx4 = _make_sc_kernel(n_i, n_j)(coord_t, embeddings_table)
    emb5 = emb4.reshape(n_j, TH, n_i // IB, JB, IB)
    emb = jnp.transpose(emb5, (2, 4, 0, 1, 3)).reshape(n_i, n_j, N_HEADS)
    idx = jnp.transpose(idx4, (1, 3, 0, 2)).reshape(n_i, n_j)
    return emb, idx


# speedup vs baseline: 1.3104x; 1.0032x over previous
import functools

import jax
import jax.numpy as jnp
from jax import lax
from jax.experimental import pallas as pl
from jax.experimental.pallas import tpu as pltpu
from jax.experimental.pallas import tpu_sc as plsc

MIN_POS = 0.0
MAX_POS = 1.0
N_POS = 100000
N_HEADS = 16

NC = 2
NS = 16
NW = NC * NS
L = 16

JB = 8
IB = 128
TH = N_HEADS // JB


def _make_sc_kernel(n_i, n_j):
    ti_count = n_i // IB
    units = (n_j // JB) * ti_count
    per_w = units // NW
    mesh = plsc.VectorSubcoreMesh(
        core_axis_name="c", subcore_axis_name="s", num_cores=NC, num_subcores=NS
    )

    @functools.partial(
        pl.kernel,
        out_type=(
            jax.ShapeDtypeStruct((n_j, TH, ti_count, JB * IB), jnp.float32),
            jax.ShapeDtypeStruct((n_j // JB, ti_count, JB, IB), jnp.int32),
        ),
        mesh=mesh,
        scratch_types=[
            pltpu.VMEM((2, JB, IB), jnp.float32),
            pltpu.VMEM((2, JB, IB), jnp.int32),
            pltpu.VMEM((2, JB * IB, N_HEADS), jnp.float32),
            pltpu.VMEM((2, JB, TH, JB * IB), jnp.float32),
            pltpu.SemaphoreType.DMA,
            pltpu.SemaphoreType.DMA,
            pltpu.SemaphoreType.DMA,
            pltpu.SemaphoreType.DMA,
            pltpu.SemaphoreType.DMA,
            pltpu.SemaphoreType.DMA,
            pltpu.SemaphoreType.DMA,
        ],
        compiler_params=pltpu.CompilerParams(
            use_tc_tiling_on_sc=False, needs_layout_passes=False
        ),
    )
    def body(
        coord_hbm, table_hbm, emb_hbm, idx_hbm,
        coord_v, idx_v, rows_v, out_v,
        sem_c, sem_g0, sem_g1, sem_i0, sem_i1, sem_o0, sem_o1,
    ):
        wid = lax.axis_index("s") * NC + lax.axis_index("c")
        wstart = wid * per_w
        scale = jnp.float32(N_POS / (MAX_POS - MIN_POS))
        lane = lax.iota(jnp.int32, L)
        hm_off = (lane & 7) * IB
        thv = lane >> 3
        jmv = [jnp.full((L,), jm, jnp.int32) for jm in range(JB)]
        sem_g = (sem_g0, sem_g1)
        sem_i = (sem_i0, sem_i1)
        sem_o = (sem_o0, sem_o1)

        def unit_pos(urel):
            g = wstart + urel
            j8 = g // ti_count
            ti = g % ti_count
            return pl.multiple_of(j8 * JB, JB), j8, ti

        def compute_idx_and_fire(urel, b):
            _, j8, ti = unit_pos(urel)
            for jm in range(JB):
                for c in range(IB // L):
                    v = coord_v[b, jm, pl.ds(c * L, L)]
                    pos = jnp.clip((v - MIN_POS) * scale, 0.0, float(N_POS))
                    idx_v[b, jm, pl.ds(c * L, L)] = pos.astype(jnp.int32)
            pltpu.async_copy(idx_v.at[b], idx_hbm.at[j8, ti], sem_i[b])
            for jm in range(JB):
                pltpu.async_copy(
                    table_hbm.at[idx_v.at[b, jm]],
                    rows_v.at[b, pl.ds(jm * IB, IB)],
                    sem_g[b],
                )

        def fire_coord(urel, b):
            j0, _, ti = unit_pos(urel)
            i0 = pl.multiple_of(ti * IB, IB)
            pltpu.async_copy(
                coord_hbm.at[pl.ds(j0, JB), pl.ds(i0, IB)],
                coord_v.at[b],
                sem_c,
            )

        def drain(src, dst, sem):
            pltpu.make_async_copy(src, dst, sem).wait()

        def drain_coord(b):
            drain(coord_hbm.at[pl.ds(0, JB), pl.ds(0, IB)], coord_v.at[b], sem_c)

        def drain_idx(b):
            drain(idx_v.at[b], idx_hbm.at[0, 0], sem_i[b])

        def drain_gathers(b):
            drain(table_hbm.at[pl.ds(0, JB * IB)], rows_v.at[b], sem_g[b])

        def drain_out(b):
            drain(out_v.at[b], emb_hbm.at[pl.ds(0, JB), :, 0], sem_o[b])

        def transpose_and_fire_out(urel, b):
            rows_b = rows_v.at[b]
            out_b = out_v.at[b]

            def tbody(ii, tc):
                off = hm_off + ii
                vecs = [rows_b[jm * IB + ii] for jm in range(JB)]
                for jm in range(JB):
                    plsc.store_scatter(out_b, [jmv[jm], thv, off], vecs[jm])
                return tc

            lax.fori_loop(0, IB, tbody, 0, unroll=2)

            j0, _, ti = unit_pos(urel)
            pltpu.async_copy(out_b, emb_hbm.at[pl.ds(j0, JB), :, ti], sem_o[b])

        j0p, _, tip = unit_pos(0)
        pltpu.sync_copy(
            coord_hbm.at[pl.ds(j0p, JB), pl.ds(pl.multiple_of(tip * IB, IB), IB)],
            coord_v.at[0],
        )
        compute_idx_and_fire(0, 0)
        fire_coord(1, 1)

        def stage(t, u, cur, is_a):
            nxt = 1 - cur
            u1 = u + 1

            def front_end():
                drain_coord(nxt)
                if is_a:
                    @pl.when(t >= 1)
                    def _():
                        drain_idx(nxt)
                else:
                    drain_idx(nxt)
                compute_idx_and_fire(u1, nxt)

                @pl.when(u + 2 < per_w)
                def _():
                    fire_coord(u + 2, cur)

            if is_a:
                front_end()
            else:
                @pl.when(u1 < per_w)
                def _():
                    front_end()

            @pl.when(t >= 1)
            def _():
                drain_out(cur)
            drain_gathers(cur)
            transpose_and_fire_out(u, cur)

        def pair_body(t, carry):
            stage(t, 2 * t, 0, True)
            stage(t, 2 * t + 1, 1, False)
            return carry

        lax.fori_loop(0, per_w // 2, pair_body, 0, unroll=False)

        drain_out(0)
        drain_out(1)
        drain_idx(0)
        drain_idx(1)

    return body


def kernel(coord, embeddings_table):
    n_i, n_j = coord.shape
    coord_t = coord.T
    emb4, idx4 = _make_sc_kernel(n_i, n_j)(coord_t, embeddings_table)
    emb5 = emb4.reshape(n_j, TH, n_i // IB, JB, IB)
    emb = jnp.transpose(emb5, (2, 4, 0, 1, 3)).reshape(n_i, n_j, N_HEADS)
    idx = jnp.transpose(idx4, (1, 3, 0, 2)).reshape(n_i, n_j)
    return emb, idx


# speedup vs baseline: 1.3164x; 1.0046x over previous
import functools

import jax
import jax.numpy as jnp
from jax import lax
from jax.experimental import pallas as pl
from jax.experimental.pallas import tpu as pltpu
from jax.experimental.pallas import tpu_sc as plsc

MIN_POS = 0.0
MAX_POS = 1.0
N_POS = 100000
N_HEADS = 16

NC = 2
NS = 16
NW = NC * NS
L = 16

JB = 8
IB = 128
TH = N_HEADS // JB


def _make_sc_kernel(n_i, n_j):
    ti_count = n_i // IB
    units = (n_j // JB) * ti_count
    per_w = units // NW
    mesh = plsc.VectorSubcoreMesh(
        core_axis_name="c", subcore_axis_name="s", num_cores=NC, num_subcores=NS
    )

    @functools.partial(
        pl.kernel,
        out_type=(
            jax.ShapeDtypeStruct((n_j, TH, ti_count, JB * IB), jnp.float32),
            jax.ShapeDtypeStruct((n_j // JB, ti_count, JB, IB), jnp.int32),
        ),
        mesh=mesh,
        scratch_types=[
            pltpu.VMEM((2, JB, IB), jnp.float32),
            pltpu.VMEM((2, JB, IB), jnp.int32),
            pltpu.VMEM((4, JB * IB, N_HEADS), jnp.float32),
            pltpu.VMEM((2, JB, TH, JB * IB), jnp.float32),
            pltpu.SemaphoreType.DMA,
            pltpu.SemaphoreType.DMA,
            pltpu.SemaphoreType.DMA,
            pltpu.SemaphoreType.DMA,
            pltpu.SemaphoreType.DMA,
            pltpu.SemaphoreType.DMA,
            pltpu.SemaphoreType.DMA,
        ],
        compiler_params=pltpu.CompilerParams(
            use_tc_tiling_on_sc=False, needs_layout_passes=False
        ),
    )
    def body(
        coord_hbm, table_hbm, emb_hbm, idx_hbm,
        coord_v, idx_v, rows_v, out_v,
        sem_c, sem_g0, sem_g1, sem_i0, sem_i1, sem_o0, sem_o1,
    ):
        wid = lax.axis_index("s") * NC + lax.axis_index("c")
        wstart = wid * per_w
        scale = jnp.float32(N_POS / (MAX_POS - MIN_POS))
        lane = lax.iota(jnp.int32, L)
        hm_off = (lane & 7) * IB
        thv = lane >> 3
        jmv = [jnp.full((L,), jm, jnp.int32) for jm in range(JB)]
        sem_g = (sem_g0, sem_g1)
        sem_i = (sem_i0, sem_i1)
        sem_o = (sem_o0, sem_o1)

        def unit_pos(urel):
            g = wstart + urel
            j8 = g // ti_count
            ti = g % ti_count
            return pl.multiple_of(j8 * JB, JB), j8, ti

        def compute_idx_and_fire(urel, p, rb):
            _, j8, ti = unit_pos(urel)
            for jm in range(JB):
                for c in range(IB // L):
                    v = coord_v[p, jm, pl.ds(c * L, L)]
                    pos = jnp.clip((v - MIN_POS) * scale, 0.0, float(N_POS))
                    idx_v[p, jm, pl.ds(c * L, L)] = pos.astype(jnp.int32)
            pltpu.async_copy(idx_v.at[p], idx_hbm.at[j8, ti], sem_i[p])
            for jm in range(JB):
                pltpu.async_copy(
                    table_hbm.at[idx_v.at[p, jm]],
                    rows_v.at[rb, pl.ds(jm * IB, IB)],
                    sem_g[p],
                )

        def fire_coord(urel, p):
            j0, _, ti = unit_pos(urel)
            i0 = pl.multiple_of(ti * IB, IB)
            pltpu.async_copy(
                coord_hbm.at[pl.ds(j0, JB), pl.ds(i0, IB)],
                coord_v.at[p],
                sem_c,
            )

        def drain(src, dst, sem):
            pltpu.make_async_copy(src, dst, sem).wait()

        def drain_coord(p):
            drain(coord_hbm.at[pl.ds(0, JB), pl.ds(0, IB)], coord_v.at[p], sem_c)

        def drain_idx(p):
            drain(idx_v.at[p], idx_hbm.at[0, 0], sem_i[p])

        def drain_gathers(p, rb):
            drain(table_hbm.at[pl.ds(0, JB * IB)], rows_v.at[rb], sem_g[p])

        def drain_out(p):
            drain(out_v.at[p], emb_hbm.at[pl.ds(0, JB), :, 0], sem_o[p])

        def transpose_and_fire_out(urel, p, rb):
            rows_b = rows_v.at[rb]
            out_b = out_v.at[p]

            def tbody(ii, tc):
                off = hm_off + ii
                vecs = [rows_b[jm * IB + ii] for jm in range(JB)]
                for jm in range(JB):
                    plsc.store_scatter(out_b, [jmv[jm], thv, off], vecs[jm])
                return tc

            lax.fori_loop(0, IB, tbody, 0, unroll=2)
            j0, _, ti = unit_pos(urel)
            pltpu.async_copy(out_b, emb_hbm.at[pl.ds(j0, JB), :, ti], sem_o[p])

        def load_coord_sync(urel, p):
            j0, _, ti = unit_pos(urel)
            pltpu.sync_copy(
                coord_hbm.at[pl.ds(j0, JB), pl.ds(pl.multiple_of(ti * IB, IB), IB)],
                coord_v.at[p],
            )

        load_coord_sync(0, 0)
        compute_idx_and_fire(0, 0, 0)
        load_coord_sync(1, 1)
        compute_idx_and_fire(1, 1, 1)
        fire_coord(2, 0)

        def stage(u, k):
            p = k % 2
            rb = k % 4
            drain_gathers(p, rb)

            @pl.when(u + 2 < per_w)
            def _():
                drain_coord(p)
                drain_idx(p)
                compute_idx_and_fire(u + 2, p, (k + 2) % 4)

                @pl.when(u + 3 < per_w)
                def _():
                    fire_coord(u + 3, (k + 3) % 2)

            @pl.when(u >= 2)
            def _():
                drain_out(p)

            transpose_and_fire_out(u, p, rb)

        def quad_body(t, carry):
            for k in range(4):
                stage(4 * t + k, k)
            return carry

        lax.fori_loop(0, per_w // 4, quad_body, 0, unroll=False)

        drain_out(0)
        drain_out(1)
        drain_idx(0)
        drain_idx(1)

    return body


def kernel(coord, embeddings_table):
    n_i, n_j = coord.shape
    coord_t = coord.T
    emb4, idx4 = _make_sc_kernel(n_i, n_j)(coord_t, embeddings_table)
    emb5 = emb4.reshape(n_j, TH, n_i // IB, JB, IB)
    emb = jnp.transpose(emb5, (2, 4, 0, 1, 3)).reshape(n_i, n_j, N_HEADS)
    idx = jnp.transpose(idx4, (1, 3, 0, 2)).reshape(n_i, n_j)
    return emb, idx
